# Initial kernel scaffold; baseline (speedup 1.0000x reference)
import jax
import jax.numpy as jnp
from jax.experimental import pallas as pl


def kernel(x, edge_index, batch, W1, b1, g1, be1, W2, b2, g2, be2, Wl1, bl1, Wl2, bl2):
    raise NotImplementedError("write your pallas kernel here")


# speedup vs baseline: 2.9436x; 2.9436x over previous
import functools

import jax
import jax.numpy as jnp
from jax import lax
from jax.experimental import pallas as pl
from jax.experimental.pallas import tpu as pltpu
import jax.experimental.pallas.tpu_sc as plsc

B_CELLS = 64


def _stage_a_body(x_ref, wa_ref, wc_ref, ca_ref, a_ref, c_ref):
    xb = x_ref[...]
    nrm = jnp.sqrt(jnp.sum(xb * xb, axis=1, keepdims=True)) + 1e-12
    h = xb / nrm
    a_ref[...] = (
        jnp.dot(h, wa_ref[...], preferred_element_type=jnp.float32) + ca_ref[...]
    )
    c_ref[...] = jnp.dot(h, wc_ref[...], preferred_element_type=jnp.float32)


def _stage_a(x, wa, wc, ca, blk):
    n, d = x.shape
    grid = n // blk
    return pl.pallas_call(
        _stage_a_body,
        grid=(grid,),
        in_specs=[
            pl.BlockSpec((blk, d), lambda i: (i, 0)),
            pl.BlockSpec((d, d), lambda i: (0, 0)),
            pl.BlockSpec((d, d), lambda i: (0, 0)),
            pl.BlockSpec((1, d), lambda i: (0, 0)),
        ],
        out_specs=[
            pl.BlockSpec((blk, d), lambda i: (i, 0)),
            pl.BlockSpec((blk, d), lambda i: (i, 0)),
        ],
        out_shape=[
            jax.ShapeDtypeStruct((n, d), jnp.float32),
            jax.ShapeDtypeStruct((n, d), jnp.float32),
        ],
    )(x, wa, wc, ca)


def _stage_b(a2, c2, batch, dst, src, *, epw, ch):
    n, d = a2.shape
    e = dst.shape[0]
    nch = epw // ch
    mesh = plsc.VectorSubcoreMesh(core_axis_name="c", subcore_axis_name="s")

    @functools.partial(
        pl.kernel,
        out_type=[
            jax.ShapeDtypeStruct((e, d), jnp.float32),
            jax.ShapeDtypeStruct((e,), jnp.int32),
        ],
        mesh=mesh,
        scratch_types=[
            pltpu.VMEM((epw,), jnp.int32),
            pltpu.VMEM((epw,), jnp.int32),
            pltpu.VMEM((ch, d), jnp.float32),
            pltpu.VMEM((ch, d), jnp.float32),
            pltpu.VMEM((ch,), jnp.int32),
            pltpu.SemaphoreType.DMA,
            pltpu.SemaphoreType.DMA,
            pltpu.SemaphoreType.DMA,
        ],
    )
    def k(a2_h, c2_h, batch_h, dst_h, src_h, r_h, cb_h,
          idx_d, idx_s, abuf, cbuf, cbv, sem0, sem1, sem2):
        wid = lax.axis_index("s") * 2 + lax.axis_index("c")
        ebase = wid * epw
        pltpu.sync_copy(dst_h.at[pl.ds(ebase, epw)], idx_d)
        pltpu.sync_copy(src_h.at[pl.ds(ebase, epw)], idx_s)

        @pl.loop(0, nch)
        def _chunk(kk):
            i0 = kk * ch
            di = idx_d.at[pl.ds(i0, ch)]
            si = idx_s.at[pl.ds(i0, ch)]
            d1 = pltpu.async_copy(a2_h.at[di], abuf, sem0)
            d2 = pltpu.async_copy(c2_h.at[si], cbuf, sem1)
            d3 = pltpu.async_copy(batch_h.at[di], cbv, sem2)
            d1.wait()
            d2.wait()

            @pl.loop(0, ch)
            def _edge(ee):
                for cc in range(d // 16):
                    av = abuf[ee, pl.ds(cc * 16, 16)]
                    cv = cbuf[ee, pl.ds(cc * 16, 16)]
                    abuf[ee, pl.ds(cc * 16, 16)] = jnp.maximum(av + cv, 0.0)

            d3.wait()
            pltpu.sync_copy(abuf, r_h.at[pl.ds(ebase + i0, ch)])
            pltpu.sync_copy(cbv, cb_h.at[pl.ds(ebase + i0, ch)])

    return k(a2, c2, batch, dst, src)


def _stage_c_body(r_ref, w_ref, b_ref, s_ref):
    s_ref[...] = jnp.maximum(
        jnp.dot(r_ref[...], w_ref[...], preferred_element_type=jnp.float32)
        + b_ref[...],
        0.0,
    )


def _stage_c(r, w2p, b2p, blk):
    e, d = r.shape
    grid = e // blk
    return pl.pallas_call(
        _stage_c_body,
        grid=(grid,),
        in_specs=[
            pl.BlockSpec((blk, d), lambda i: (i, 0)),
            pl.BlockSpec((d, d), lambda i: (0, 0)),
            pl.BlockSpec((1, d), lambda i: (0, 0)),
        ],
        out_specs=pl.BlockSpec((blk, d), lambda i: (i, 0)),
        out_shape=jax.ShapeDtypeStruct((e, d), jnp.float32),
    )(r, w2p, b2p)


def _stage_d(s, cb, *, epw, ch):
    e, d = s.shape
    nch = epw // ch
    nw = 32
    mesh = plsc.VectorSubcoreMesh(core_axis_name="c", subcore_axis_name="s")

    @functools.partial(
        pl.kernel,
        out_type=jax.ShapeDtypeStruct((nw, B_CELLS, d), jnp.float32),
        mesh=mesh,
        compiler_params=pltpu.CompilerParams(needs_layout_passes=False),
        scratch_types=[
            pltpu.VMEM((ch, d), jnp.float32),
            pltpu.VMEM((ch,), jnp.int32),
            pltpu.VMEM((B_CELLS, d), jnp.float32),
        ],
    )
    def k(s_h, cb_h, out_h, sbuf, cbv, acc):
        lane = lax.iota(jnp.int32, 16)
        wid = lax.axis_index("s") * 2 + lax.axis_index("c")
        ebase = wid * epw

        @pl.loop(0, B_CELLS)
        def _zrow(rr):
            for cc in range(d // 16):
                acc[rr, pl.ds(cc * 16, 16)] = jnp.zeros((16,), jnp.float32)

        @pl.loop(0, nch)
        def _chunk(kk):
            i0 = ebase + kk * ch
            pltpu.sync_copy(s_h.at[pl.ds(i0, ch)], sbuf)
            pltpu.sync_copy(cb_h.at[pl.ds(i0, ch)], cbv)

            @pl.loop(0, ch // 16)
            def _grp(gg):
                cb16 = cbv[pl.ds(gg * 16, 16)]
                for j in range(16):
                    cj = jnp.sum(jnp.where(lane == j, cb16, 0))
                    ee = gg * 16 + j
                    for cc in range(d // 16):
                        sv = sbuf[ee, pl.ds(cc * 16, 16)]
                        av = acc[cj, pl.ds(cc * 16, 16)]
                        acc[cj, pl.ds(cc * 16, 16)] = jnp.maximum(av, sv)

        pltpu.sync_copy(acc, out_h.at[wid])

    return k(s, cb)


def _stage_e_body(p_ref, wl1_ref, bl1_ref, wl2_ref, bl2_ref, y_ref):
    pooled = jnp.max(p_ref[...], axis=0)
    y1 = jnp.maximum(
        jnp.dot(pooled, wl1_ref[...], preferred_element_type=jnp.float32)
        + bl1_ref[...],
        0.0,
    )
    y2 = (
        jnp.dot(y1, wl2_ref[...], preferred_element_type=jnp.float32) + bl2_ref[...]
    )
    nrm = jnp.sqrt(jnp.sum(y2 * y2, axis=1, keepdims=True)) + 1e-12
    y_ref[...] = y2 / nrm


def _stage_e(partials, wl1t, bl1, wl2t, bl2):
    nw, b, d = partials.shape
    return pl.pallas_call(
        _stage_e_body,
        in_specs=[
            pl.BlockSpec((nw, b, d), lambda: (0, 0, 0)),
            pl.BlockSpec((d, d), lambda: (0, 0)),
            pl.BlockSpec((1, d), lambda: (0, 0)),
            pl.BlockSpec((d, d), lambda: (0, 0)),
            pl.BlockSpec((1, d), lambda: (0, 0)),
        ],
        out_specs=pl.BlockSpec((b, d), lambda: (0, 0)),
        out_shape=jax.ShapeDtypeStruct((b, d), jnp.float32),
    )(partials, wl1t, bl1, wl2t, bl2)


def kernel(x, edge_index, batch, W1, b1, g1, be1, W2, b2, g2, be2,
           Wl1, bl1, Wl2, bl2):
    n, d = x.shape
    e = edge_index.shape[1]
    nw = 32
    epw = e // nw
    assert e % nw == 0 and n % 8 == 0

    w1a = W1[:, :d]
    w1b = W1[:, d:]
    wa = (w1a - w1b).T * g1[None, :]
    wc = w1b.T * g1[None, :]
    ca = (g1 * b1 + be1)[None, :]
    w2p = (W2 * g2[:, None]).T
    b2p = (g2 * b2 + be2)[None, :]

    a2, c2 = _stage_a(x, wa, wc, ca, blk=400)
    dst = edge_index[1]
    src = edge_index[0]
    r, cb = _stage_b(a2, c2, batch, dst, src, epw=epw, ch=80)
    s = _stage_c(r, w2p, b2p, blk=2000)
    partials = _stage_d(s, cb, epw=epw, ch=400)
    y = _stage_e(partials, Wl1.T, bl1[None, :], Wl2.T, bl2[None, :])
    return y
